# baseline (speedup 1.0000x reference)
import functools

import jax
import jax.numpy as jnp
from jax import lax
from jax.experimental import pallas as pl
from jax.experimental.pallas import tpu as pltpu
from jax.experimental.pallas import tpu_sc as plsc

VOCAB = 100000
EMBED = 64
BATCH = 16384

NUM_CORES = 2
NUM_SUBCORES = 16
LANES = 16
NW = NUM_CORES * NUM_SUBCORES
BPW = BATCH // NW
CHUNK = 128
NCHUNK = BPW // CHUNK


def _make_kernel():
  mesh = plsc.VectorSubcoreMesh(core_axis_name="c", subcore_axis_name="s")

  @functools.partial(
      pl.kernel,
      out_type=jax.ShapeDtypeStruct((BATCH, 1), jnp.float32),
      mesh=mesh,
      compiler_params=pltpu.CompilerParams(
          needs_layout_passes=False, use_tc_tiling_on_sc=False),
      scratch_types=[
          pltpu.VMEM((BPW,), jnp.int32),
          pltpu.VMEM((BPW,), jnp.int32),
          pltpu.VMEM((BPW, EMBED), jnp.float32),
          pltpu.VMEM((BPW, EMBED), jnp.float32),
          pltpu.VMEM((BPW, 1), jnp.float32),
          pltpu.SemaphoreType.DMA,
      ],
  )
  def word2vec_dots(tgt_hbm, ctx_hbm, ttab_hbm, ctab_hbm, out_hbm,
                    tidx_v, cidx_v, trow_v, crow_v, out_v, sem):
    wid = lax.axis_index("s") * NUM_CORES + lax.axis_index("c")

    pltpu.sync_copy(tgt_hbm.at[pl.ds(wid * BPW, BPW)], tidx_v)
    pltpu.sync_copy(ctx_hbm.at[pl.ds(wid * BPW, BPW)], cidx_v)

    copies = []
    for j in range(NCHUNK):
      sl = pl.ds(j * CHUNK, CHUNK)
      copies.append(pltpu.async_copy(
          ttab_hbm.at[tidx_v.at[sl]], trow_v.at[sl], sem))
      copies.append(pltpu.async_copy(
          ctab_hbm.at[cidx_v.at[sl]], crow_v.at[sl], sem))
    for c in copies:
      c.wait()

    lane = lax.iota(jnp.int32, LANES)
    zero = jnp.zeros((LANES,), jnp.int32)

    def body(i, _):
      rows = i * LANES + lane
      acc = jnp.zeros((LANES,), jnp.float32)
      for d in range(EMBED):
        col = jnp.full((LANES,), d, jnp.int32)
        tv = plsc.load_gather(trow_v, [rows, col])
        cv = plsc.load_gather(crow_v, [rows, col])
        acc = acc + tv * cv
      plsc.store_scatter(out_v, [rows, zero], acc)
      return 0

    lax.fori_loop(0, BPW // LANES, body, 0)

    pltpu.sync_copy(out_v, out_hbm.at[pl.ds(wid * BPW, BPW)])

  return word2vec_dots


_word2vec_dots = _make_kernel()


@jax.jit
def kernel(target, context, target_table, context_table):
  return _word2vec_dots(target.astype(jnp.int32), context.astype(jnp.int32),
                        target_table, context_table)


# speedup vs baseline: 1.1853x; 1.1853x over previous
import functools

import jax
import jax.numpy as jnp
from jax import lax
from jax.experimental import pallas as pl
from jax.experimental.pallas import tpu as pltpu
from jax.experimental.pallas import tpu_sc as plsc

VOCAB = 100000
EMBED = 64
BATCH = 16384

NUM_CORES = 2
NUM_SUBCORES = 16
LANES = 16
NW = NUM_CORES * NUM_SUBCORES
BPW = BATCH // NW
CHUNK = 128
NCHUNK = BPW // CHUNK


def _make_kernel():
  mesh = plsc.VectorSubcoreMesh(core_axis_name="c", subcore_axis_name="s")

  @functools.partial(
      pl.kernel,
      out_type=jax.ShapeDtypeStruct((BATCH, 1), jnp.float32),
      mesh=mesh,
      compiler_params=pltpu.CompilerParams(
          needs_layout_passes=False, use_tc_tiling_on_sc=False),
      scratch_types=[
          pltpu.VMEM((BPW,), jnp.int32),
          pltpu.VMEM((BPW,), jnp.int32),
          pltpu.VMEM((BPW, EMBED), jnp.float32),
          pltpu.VMEM((BPW, EMBED), jnp.float32),
          pltpu.VMEM((BPW, 1), jnp.float32),
          pltpu.SemaphoreType.DMA,
      ],
  )
  def word2vec_dots(tgt_hbm, ctx_hbm, ttab_hbm, ctab_hbm, out_hbm,
                    tidx_v, cidx_v, trow_v, crow_v, out_v, sem):
    wid = lax.axis_index("s") * NUM_CORES + lax.axis_index("c")

    pltpu.sync_copy(tgt_hbm.at[pl.ds(wid * BPW, BPW)], tidx_v)
    pltpu.sync_copy(ctx_hbm.at[pl.ds(wid * BPW, BPW)], cidx_v)

    copies = []
    for j in range(NCHUNK):
      sl = pl.ds(j * CHUNK, CHUNK)
      copies.append(pltpu.async_copy(
          ttab_hbm.at[tidx_v.at[sl]], trow_v.at[sl], sem))
      copies.append(pltpu.async_copy(
          ctab_hbm.at[cidx_v.at[sl]], crow_v.at[sl], sem))
    for c in copies:
      c.wait()

    lane = lax.iota(jnp.int32, LANES)
    zero = jnp.zeros((LANES,), jnp.int32)

    def body(i, _):
      rows = i * LANES + lane
      acc = jnp.zeros((LANES,), jnp.float32)
      for d in range(EMBED):
        col = jnp.bitwise_and(lane + d, EMBED - 1)
        tv = plsc.load_gather(trow_v, [rows, col])
        cv = plsc.load_gather(crow_v, [rows, col])
        acc = acc + tv * cv
      plsc.store_scatter(out_v, [rows, zero], acc)
      return 0

    lax.fori_loop(0, BPW // LANES, body, 0)

    pltpu.sync_copy(out_v, out_hbm.at[pl.ds(wid * BPW, BPW)])

  return word2vec_dots


_word2vec_dots = _make_kernel()


@jax.jit
def kernel(target, context, target_table, context_table):
  return _word2vec_dots(target.astype(jnp.int32), context.astype(jnp.int32),
                        target_table, context_table)
